# Initial kernel scaffold; baseline (speedup 1.0000x reference)
import jax
import jax.numpy as jnp
from jax.experimental import pallas as pl


def kernel(x, edge_index, batch, W1, b1, g1, be1, W2, b2, g2, be2, Wfc, bfc):
    raise NotImplementedError("write your pallas kernel here")


# speedup vs baseline: 10.8952x; 10.8952x over previous
import functools

import jax
import jax.numpy as jnp
from jax import lax
from jax.experimental import pallas as pl
from jax.experimental.pallas import tpu as pltpu
from jax.experimental.pallas import tpu_sc as plsc

N = 10000
E = 320000
D = 128
H = 128
G = 64
EPS = 1e-5

NC = 2
NS = 16
NW = NC * NS
CH = 80
NCHP = E // NS // CH
ZR = 128

RB = 400
NB = N // RB

_sc_mesh = plsc.VectorSubcoreMesh(core_axis_name="c", subcore_axis_name="s",
                                  num_cores=NC, num_subcores=NS)


NPAD2 = 5120
RP2 = NPAD2 // NS
ACC_R = NPAD2 + NS
DW = 8


def _localize(dstv, nchunks, lo, garb):
    def body(j, _):
        for k in range(CH // 16):
            v = dstv[j, pl.ds(k * 16, 16)]
            l = v - lo
            ok = (l >= 0) & (l < NPAD2)
            dstv[j, pl.ds(k * 16, 16)] = jnp.where(ok, l, garb)
        return 0

    lax.fori_loop(0, nchunks, body, 0)


def _deg_body(dst_hbm, ones_hbm, zeros_hbm, deg_hbm, dstv, onesv, zrow,
              acc_sh):
    c = lax.axis_index("c")
    s = lax.axis_index("s")

    pltpu.sync_copy(zeros_hbm, zrow)
    pltpu.sync_copy(ones_hbm, onesv)
    base = s * RP2
    pltpu.sync_copy(zrow, acc_sh.at[pl.ds(base, ZR)])
    pltpu.sync_copy(zrow, acc_sh.at[pl.ds(base + ZR, ZR)])
    pltpu.sync_copy(zrow.at[pl.ds(0, RP2 - 2 * ZR)],
                    acc_sh.at[pl.ds(base + 2 * ZR, RP2 - 2 * ZR)])

    pltpu.sync_copy(dst_hbm.at[s], dstv)
    _localize(dstv, NCHP, c * NPAD2, NPAD2 + s)
    plsc.subcore_barrier()

    def step(j, _):
        pltpu.sync_copy(onesv, acc_sh.at[dstv.at[j]], add=True)
        return 0

    lax.fori_loop(0, NCHP, step, 0)
    plsc.subcore_barrier()
    pltpu.sync_copy(acc_sh.at[pl.ds(base, RP2)],
                    deg_hbm.at[c, pl.ds(base, RP2)])


@functools.partial(
    pl.kernel,
    out_type=jax.ShapeDtypeStruct((NC, NPAD2, DW), jnp.float32),
    mesh=_sc_mesh,
    scratch_types=[
        pltpu.VMEM((NCHP, CH), jnp.int32),
        pltpu.VMEM((CH, DW), jnp.float32),
        pltpu.VMEM((ZR, DW), jnp.float32),
        pltpu.VMEM_SHARED((ACC_R, DW), jnp.float32),
    ],
)
def _deg_sc(dst_hbm, ones_hbm, zeros_hbm, deg_hbm, dstv, onesv, zrow,
            acc_sh):
    _deg_body(dst_hbm, ones_hbm, zeros_hbm, deg_hbm, dstv, onesv, zrow,
              acc_sh)


def _prop_body(u_hbm, src_hbm, dst_hbm, zeros_hbm, out_hbm, srcv, dstv,
               rows, acc_sh):
    c = lax.axis_index("c")
    s = lax.axis_index("s")

    base = s * RP2
    pltpu.sync_copy(zeros_hbm, acc_sh.at[pl.ds(base, ZR)])
    pltpu.sync_copy(zeros_hbm, acc_sh.at[pl.ds(base + ZR, ZR)])
    pltpu.sync_copy(zeros_hbm.at[pl.ds(0, RP2 - 2 * ZR)],
                    acc_sh.at[pl.ds(base + 2 * ZR, RP2 - 2 * ZR)])

    pltpu.sync_copy(src_hbm.at[s], srcv)
    pltpu.sync_copy(dst_hbm.at[s], dstv)
    _localize(dstv, NCHP, c * NPAD2, NPAD2 + s)
    plsc.subcore_barrier()

    def step(j, _):
        pltpu.sync_copy(u_hbm.at[srcv.at[j]], rows)
        pltpu.sync_copy(rows, acc_sh.at[dstv.at[j]], add=True)
        return 0

    lax.fori_loop(0, NCHP, step, 0)
    plsc.subcore_barrier()
    pltpu.sync_copy(acc_sh.at[pl.ds(s * RP2, RP2)],
                    out_hbm.at[c, pl.ds(s * RP2, RP2)])


@functools.partial(
    pl.kernel,
    out_type=jax.ShapeDtypeStruct((NC, NPAD2, H), jnp.float32),
    mesh=_sc_mesh,
    scratch_types=[
        pltpu.VMEM((NCHP, CH), jnp.int32),
        pltpu.VMEM((NCHP, CH), jnp.int32),
        pltpu.VMEM((CH, H), jnp.float32),
        pltpu.VMEM_SHARED((ACC_R, H), jnp.float32),
    ],
)
def _prop_sc(u_hbm, src_hbm, dst_hbm, zeros_hbm, out_hbm, srcv, dstv,
             rows, acc_sh):
    _prop_body(u_hbm, src_hbm, dst_hbm, zeros_hbm, out_hbm, srcv, dstv,
               rows, acc_sh)


def _dinv_from_deg(degc_ref):
    deg = degc_ref[...] + 1.0
    return lax.rsqrt(jnp.maximum(deg, 1.0))


def _mm1_kern(x_ref, w_ref, o_ref):
    o_ref[...] = jnp.dot(x_ref[...], w_ref[...],
                         preferred_element_type=jnp.float32)


def _mm1(x, W1):
    return pl.pallas_call(
        _mm1_kern,
        grid=(NB,),
        in_specs=[pl.BlockSpec((RB, D), lambda i: (i, 0)),
                  pl.BlockSpec((D, H), lambda i: (0, 0))],
        out_specs=pl.BlockSpec((RB, H), lambda i: (i, 0)),
        out_shape=jax.ShapeDtypeStruct((N, H), jnp.float32),
    )(x, W1)


def _scaleu_kern(degp_ref, xw_ref, o_ref):
    o_ref[...] = xw_ref[...] * _dinv_from_deg(degp_ref)


def _scaleu(degP, xW):
    return pl.pallas_call(
        _scaleu_kern,
        grid=(NB,),
        in_specs=[pl.BlockSpec((RB, 1), lambda i: (i, 0)),
                  pl.BlockSpec((RB, H), lambda i: (i, 0))],
        out_specs=pl.BlockSpec((RB, H), lambda i: (i, 0)),
        out_shape=jax.ShapeDtypeStruct((N, H), jnp.float32),
    )(degP, xW)


def _a1_kern(p_ref, u_ref, degp_ref, b_ref, a_ref, st_ref):
    dinv = _dinv_from_deg(degp_ref)
    tot = p_ref[...] + u_ref[...]
    a = jnp.maximum(tot * dinv + b_ref[...], 0.0)
    a_ref[...] = a

    @pl.when(pl.program_id(0) == 0)
    def _():
        st_ref[...] = jnp.zeros_like(st_ref)

    st_ref[...] += jnp.concatenate(
        [jnp.sum(a, axis=0, keepdims=True),
         jnp.sum(a * a, axis=0, keepdims=True)], axis=0)


def _a1(P, u1, degP, b1r):
    return pl.pallas_call(
        _a1_kern,
        grid=(NB,),
        in_specs=[pl.BlockSpec((RB, H), lambda i: (i, 0)),
                  pl.BlockSpec((RB, H), lambda i: (i, 0)),
                  pl.BlockSpec((RB, 1), lambda i: (i, 0)),
                  pl.BlockSpec((1, H), lambda i: (0, 0))],
        out_specs=[pl.BlockSpec((RB, H), lambda i: (i, 0)),
                   pl.BlockSpec((2, H), lambda i: (0, 0))],
        out_shape=[jax.ShapeDtypeStruct((N, H), jnp.float32),
                   jax.ShapeDtypeStruct((2, H), jnp.float32)],
    )(P, u1, degP, b1r)


def _bn_coeffs(st_ref, g_ref, be_ref):
    mean = st_ref[0:1, :] * (1.0 / N)
    var = st_ref[1:2, :] * (1.0 / N) - mean * mean
    scale = g_ref[...] * lax.rsqrt(var + EPS)
    shift = be_ref[...] - mean * scale
    return scale, shift


def _m2_kern(a_ref, degp_ref, st_ref, g_ref, be_ref, w_ref, o_ref):
    scale, shift = _bn_coeffs(st_ref, g_ref, be_ref)
    t = a_ref[...] * scale + shift
    o_ref[...] = jnp.dot(t, w_ref[...],
                         preferred_element_type=jnp.float32) \
        * _dinv_from_deg(degp_ref)


def _m2(a1, degP, stats1, g1r, be1r, W2):
    return pl.pallas_call(
        _m2_kern,
        grid=(NB,),
        in_specs=[pl.BlockSpec((RB, H), lambda i: (i, 0)),
                  pl.BlockSpec((RB, 1), lambda i: (i, 0)),
                  pl.BlockSpec((2, H), lambda i: (0, 0)),
                  pl.BlockSpec((1, H), lambda i: (0, 0)),
                  pl.BlockSpec((1, H), lambda i: (0, 0)),
                  pl.BlockSpec((H, H), lambda i: (0, 0))],
        out_specs=pl.BlockSpec((RB, H), lambda i: (i, 0)),
        out_shape=jax.ShapeDtypeStruct((N, H), jnp.float32),
    )(a1, degP, stats1, g1r, be1r, W2)


def _a2_kern(q_ref, u_ref, degp_ref, b_ref, batch_ref,
             s_ref, c_ref, st_ref):
    dinv = _dinv_from_deg(degp_ref)
    tot = q_ref[...] + u_ref[...]
    a = jnp.maximum(tot * dinv + b_ref[...], 0.0)

    bvals = batch_ref[0, 0, :]
    gids = lax.broadcasted_iota(jnp.int32, (RB, G), 1)
    onehot = (bvals[:, None] == gids).astype(jnp.float32)

    @pl.when(pl.program_id(0) == 0)
    def _():
        s_ref[...] = jnp.zeros_like(s_ref)
        c_ref[...] = jnp.zeros_like(c_ref)
        st_ref[...] = jnp.zeros_like(st_ref)

    dn = (((0,), (0,)), ((), ()))
    s_ref[...] += lax.dot_general(onehot, a, dn,
                                  preferred_element_type=jnp.float32)
    c_ref[...] += lax.dot_general(onehot, jnp.ones_like(a), dn,
                                  preferred_element_type=jnp.float32)
    st_ref[...] += jnp.concatenate(
        [jnp.sum(a, axis=0, keepdims=True),
         jnp.sum(a * a, axis=0, keepdims=True)], axis=0)


def _a2(Q, u2, degP, b2r, batch_r):
    return pl.pallas_call(
        _a2_kern,
        grid=(NB,),
        in_specs=[pl.BlockSpec((RB, H), lambda i: (i, 0)),
                  pl.BlockSpec((RB, H), lambda i: (i, 0)),
                  pl.BlockSpec((RB, 1), lambda i: (i, 0)),
                  pl.BlockSpec((1, H), lambda i: (0, 0)),
                  pl.BlockSpec((1, 1, RB), lambda i: (i, 0, 0))],
        out_specs=[pl.BlockSpec((G, H), lambda i: (0, 0)),
                   pl.BlockSpec((G, H), lambda i: (0, 0)),
                   pl.BlockSpec((2, H), lambda i: (0, 0))],
        out_shape=[jax.ShapeDtypeStruct((G, H), jnp.float32),
                   jax.ShapeDtypeStruct((G, H), jnp.float32),
                   jax.ShapeDtypeStruct((2, H), jnp.float32)],
    )(Q, u2, degP, b2r, batch_r)


def _final_kern(s_ref, c_ref, st_ref, g_ref, be_ref, wfc_ref, bfc_ref,
                o_ref):
    scale, shift = _bn_coeffs(st_ref, g_ref, be_ref)
    cnt = c_ref[...]
    pooled_raw = s_ref[...] / jnp.maximum(cnt, 1.0)
    pooled = jnp.where(cnt > 0.0, pooled_raw * scale + shift, 0.0)
    z = jnp.sum(pooled * wfc_ref[...], axis=1, keepdims=True) \
        + bfc_ref[0, 0]
    o_ref[...] = 1.0 / (1.0 + jnp.exp(-z))


def _final(S, C, stats2, g2r, be2r, wfcr, bfcr):
    return pl.pallas_call(
        _final_kern,
        in_specs=[pl.BlockSpec((G, H), lambda: (0, 0)),
                  pl.BlockSpec((G, H), lambda: (0, 0)),
                  pl.BlockSpec((2, H), lambda: (0, 0)),
                  pl.BlockSpec((1, H), lambda: (0, 0)),
                  pl.BlockSpec((1, H), lambda: (0, 0)),
                  pl.BlockSpec((1, H), lambda: (0, 0)),
                  pl.BlockSpec((1, 1), lambda: (0, 0))],
        out_specs=pl.BlockSpec((G, 1), lambda: (0, 0)),
        out_shape=jax.ShapeDtypeStruct((G, 1), jnp.float32),
    )(S, C, stats2, g2r, be2r, wfcr, bfcr)


def kernel(x, edge_index, batch, W1, b1, g1, be1, W2, b2, g2, be2, Wfc, bfc):
    src32 = edge_index[0].astype(jnp.int32)
    dst32 = edge_index[1].astype(jnp.int32)
    src_p = src32.reshape(NS, NCHP, CH)
    dst_p = dst32.reshape(NS, NCHP, CH)
    batch_r = batch.astype(jnp.int32).reshape(NB, 1, RB)
    b1r = b1.reshape(1, H)
    g1r = g1.reshape(1, H)
    be1r = be1.reshape(1, H)
    b2r = b2.reshape(1, H)
    g2r = g2.reshape(1, H)
    be2r = be2.reshape(1, H)
    wfcr = Wfc.reshape(1, H)
    bfcr = bfc.reshape(1, 1)

    zerosH = jnp.zeros((ZR, H), jnp.float32)
    onesD = jnp.ones((CH, DW), jnp.float32)
    zerosD = jnp.zeros((ZR, DW), jnp.float32)

    degc = _deg_sc(dst_p, onesD, zerosD).reshape(NC * NPAD2, DW)[:, :1]
    xW = _mm1(x, W1)
    u1 = _scaleu(degc, xW)
    P = _prop_sc(u1, src_p, dst_p, zerosH).reshape(NC * NPAD2, H)
    a1, stats1 = _a1(P, u1, degc, b1r)
    u2 = _m2(a1, degc, stats1, g1r, be1r, W2)
    Q = _prop_sc(u2, src_p, dst_p, zerosH).reshape(NC * NPAD2, H)
    S, C, stats2 = _a2(Q, u2, degc, b2r, batch_r)
    return _final(S, C, stats2, g2r, be2r, wfcr, bfcr)


# speedup vs baseline: 17.2849x; 1.5865x over previous
import functools

import jax
import jax.numpy as jnp
from jax import lax
from jax.experimental import pallas as pl
from jax.experimental.pallas import tpu as pltpu
from jax.experimental.pallas import tpu_sc as plsc

N = 10000
E = 320000
D = 128
H = 128
G = 64
EPS = 1e-5

NC = 2
NS = 16
NW = NC * NS
CH = 80
NCHP = E // NS // CH
ZR = 128

RB = 400
NB = N // RB

_sc_mesh = plsc.VectorSubcoreMesh(core_axis_name="c", subcore_axis_name="s",
                                  num_cores=NC, num_subcores=NS)


NPAD2 = 5120
RP2 = NPAD2 // NS
ACC_R = NPAD2 + NS
DW = 8


def _localize(dstv, nchunks, lo, garb):
    def body(j, _):
        for k in range(CH // 16):
            v = dstv[j, pl.ds(k * 16, 16)]
            l = v - lo
            ok = (l >= 0) & (l < NPAD2)
            dstv[j, pl.ds(k * 16, 16)] = jnp.where(ok, l, garb)
        return 0

    lax.fori_loop(0, nchunks, body, 0)


def _deg_body(dst_hbm, ones_hbm, zeros_hbm, deg_hbm, dstv, onesv, zrow,
              acc_sh, dsem):
    c = lax.axis_index("c")
    s = lax.axis_index("s")

    pltpu.sync_copy(zeros_hbm, zrow)
    pltpu.sync_copy(ones_hbm, onesv)
    base = s * RP2
    pltpu.sync_copy(zrow, acc_sh.at[pl.ds(base, ZR)])
    pltpu.sync_copy(zrow, acc_sh.at[pl.ds(base + ZR, ZR)])
    pltpu.sync_copy(zrow.at[pl.ds(0, RP2 - 2 * ZR)],
                    acc_sh.at[pl.ds(base + 2 * ZR, RP2 - 2 * ZR)])

    pltpu.sync_copy(dst_hbm.at[s], dstv)
    _localize(dstv, NCHP, c * NPAD2, NPAD2 + s)
    plsc.subcore_barrier()

    def step(j, _):
        pltpu.sync_copy(onesv, acc_sh.at[dstv.at[j]], add=True)
        return 0

    lax.fori_loop(0, NCHP, step, 0)
    plsc.subcore_barrier()
    pltpu.sync_copy(acc_sh.at[pl.ds(base, RP2)],
                    deg_hbm.at[c, pl.ds(base, RP2)])


@functools.partial(
    pl.kernel,
    out_type=jax.ShapeDtypeStruct((NC, NPAD2, DW), jnp.float32),
    mesh=_sc_mesh,
    scratch_types=[
        pltpu.VMEM((NCHP, CH), jnp.int32),
        pltpu.VMEM((CH, DW), jnp.float32),
        pltpu.VMEM((ZR, DW), jnp.float32),
        pltpu.VMEM_SHARED((ACC_R, DW), jnp.float32),
        pltpu.SemaphoreType.DMA,
    ],
)
def _deg_sc(dst_hbm, ones_hbm, zeros_hbm, deg_hbm, dstv, onesv, zrow,
            acc_sh, dsem):
    _deg_body(dst_hbm, ones_hbm, zeros_hbm, deg_hbm, dstv, onesv, zrow,
              acc_sh, dsem)


NBUF = 2


def _prop_body(u_hbm, src_hbm, dst_hbm, zeros_hbm, out_hbm, srcv, dstv,
               rows, acc_sh, gsem):
    c = lax.axis_index("c")
    s = lax.axis_index("s")

    base = s * RP2
    pltpu.sync_copy(zeros_hbm, acc_sh.at[pl.ds(base, ZR)])
    pltpu.sync_copy(zeros_hbm, acc_sh.at[pl.ds(base + ZR, ZR)])
    pltpu.sync_copy(zeros_hbm.at[pl.ds(0, RP2 - 2 * ZR)],
                    acc_sh.at[pl.ds(base + 2 * ZR, RP2 - 2 * ZR)])

    pltpu.sync_copy(src_hbm.at[s], srcv)
    pltpu.sync_copy(dst_hbm.at[s], dstv)
    _localize(dstv, NCHP, c * NPAD2, NPAD2 + s)
    plsc.subcore_barrier()

    for b in range(NBUF):
        pltpu.async_copy(u_hbm.at[srcv.at[b]], rows.at[b], gsem.at[b])

    def outer(g, _):
        for b in range(NBUF):
            j = g * NBUF + b
            pltpu.make_async_copy(u_hbm.at[srcv.at[j]], rows.at[b],
                                  gsem.at[b]).wait()
            pltpu.sync_copy(rows.at[b], acc_sh.at[dstv.at[j]], add=True)
            nxt = j + NBUF

            @pl.when(nxt < NCHP)
            def _():
                pltpu.async_copy(u_hbm.at[srcv.at[nxt]], rows.at[b],
                                 gsem.at[b])
        return 0

    lax.fori_loop(0, NCHP // NBUF, outer, 0)
    plsc.subcore_barrier()
    pltpu.sync_copy(acc_sh.at[pl.ds(s * RP2, RP2)],
                    out_hbm.at[c, pl.ds(s * RP2, RP2)])


@functools.partial(
    pl.kernel,
    out_type=jax.ShapeDtypeStruct((NC, NPAD2, H), jnp.float32),
    mesh=_sc_mesh,
    scratch_types=[
        pltpu.VMEM((NCHP, CH), jnp.int32),
        pltpu.VMEM((NCHP, CH), jnp.int32),
        pltpu.VMEM((NBUF, CH, H), jnp.float32),
        pltpu.VMEM_SHARED((ACC_R, H), jnp.float32),
        pltpu.SemaphoreType.DMA((NBUF,)),
    ],
)
def _prop_sc(u_hbm, src_hbm, dst_hbm, zeros_hbm, out_hbm, srcv, dstv,
             rows, acc_sh, gsem):
    _prop_body(u_hbm, src_hbm, dst_hbm, zeros_hbm, out_hbm, srcv, dstv,
               rows, acc_sh, gsem)


def _dinv_from_deg(degc_ref):
    deg = degc_ref[...] + 1.0
    return lax.rsqrt(jnp.maximum(deg, 1.0))


def _mm1_kern(x_ref, w_ref, o_ref):
    o_ref[...] = jnp.dot(x_ref[...], w_ref[...],
                         preferred_element_type=jnp.float32)


def _mm1(x, W1):
    return pl.pallas_call(
        _mm1_kern,
        grid=(NB,),
        in_specs=[pl.BlockSpec((RB, D), lambda i: (i, 0)),
                  pl.BlockSpec((D, H), lambda i: (0, 0))],
        out_specs=pl.BlockSpec((RB, H), lambda i: (i, 0)),
        out_shape=jax.ShapeDtypeStruct((N, H), jnp.float32),
    )(x, W1)


def _scaleu_kern(degp_ref, xw_ref, o_ref):
    o_ref[...] = xw_ref[...] * _dinv_from_deg(degp_ref)


def _scaleu(degP, xW):
    return pl.pallas_call(
        _scaleu_kern,
        grid=(NB,),
        in_specs=[pl.BlockSpec((RB, 1), lambda i: (i, 0)),
                  pl.BlockSpec((RB, H), lambda i: (i, 0))],
        out_specs=pl.BlockSpec((RB, H), lambda i: (i, 0)),
        out_shape=jax.ShapeDtypeStruct((N, H), jnp.float32),
    )(degP, xW)


def _a1_kern(p_ref, u_ref, degp_ref, b_ref, a_ref, st_ref):
    dinv = _dinv_from_deg(degp_ref)
    tot = p_ref[...] + u_ref[...]
    a = jnp.maximum(tot * dinv + b_ref[...], 0.0)
    a_ref[...] = a

    @pl.when(pl.program_id(0) == 0)
    def _():
        st_ref[...] = jnp.zeros_like(st_ref)

    st_ref[...] += jnp.concatenate(
        [jnp.sum(a, axis=0, keepdims=True),
         jnp.sum(a * a, axis=0, keepdims=True)], axis=0)


def _a1(P, u1, degP, b1r):
    return pl.pallas_call(
        _a1_kern,
        grid=(NB,),
        in_specs=[pl.BlockSpec((RB, H), lambda i: (i, 0)),
                  pl.BlockSpec((RB, H), lambda i: (i, 0)),
                  pl.BlockSpec((RB, 1), lambda i: (i, 0)),
                  pl.BlockSpec((1, H), lambda i: (0, 0))],
        out_specs=[pl.BlockSpec((RB, H), lambda i: (i, 0)),
                   pl.BlockSpec((2, H), lambda i: (0, 0))],
        out_shape=[jax.ShapeDtypeStruct((N, H), jnp.float32),
                   jax.ShapeDtypeStruct((2, H), jnp.float32)],
    )(P, u1, degP, b1r)


def _bn_coeffs(st_ref, g_ref, be_ref):
    mean = st_ref[0:1, :] * (1.0 / N)
    var = st_ref[1:2, :] * (1.0 / N) - mean * mean
    scale = g_ref[...] * lax.rsqrt(var + EPS)
    shift = be_ref[...] - mean * scale
    return scale, shift


def _m2_kern(a_ref, degp_ref, st_ref, g_ref, be_ref, w_ref, o_ref):
    scale, shift = _bn_coeffs(st_ref, g_ref, be_ref)
    t = a_ref[...] * scale + shift
    o_ref[...] = jnp.dot(t, w_ref[...],
                         preferred_element_type=jnp.float32) \
        * _dinv_from_deg(degp_ref)


def _m2(a1, degP, stats1, g1r, be1r, W2):
    return pl.pallas_call(
        _m2_kern,
        grid=(NB,),
        in_specs=[pl.BlockSpec((RB, H), lambda i: (i, 0)),
                  pl.BlockSpec((RB, 1), lambda i: (i, 0)),
                  pl.BlockSpec((2, H), lambda i: (0, 0)),
                  pl.BlockSpec((1, H), lambda i: (0, 0)),
                  pl.BlockSpec((1, H), lambda i: (0, 0)),
                  pl.BlockSpec((H, H), lambda i: (0, 0))],
        out_specs=pl.BlockSpec((RB, H), lambda i: (i, 0)),
        out_shape=jax.ShapeDtypeStruct((N, H), jnp.float32),
    )(a1, degP, stats1, g1r, be1r, W2)


def _a2_kern(q_ref, u_ref, degp_ref, b_ref, batch_ref,
             s_ref, c_ref, st_ref):
    dinv = _dinv_from_deg(degp_ref)
    tot = q_ref[...] + u_ref[...]
    a = jnp.maximum(tot * dinv + b_ref[...], 0.0)

    bvals = batch_ref[0, 0, :]
    gids = lax.broadcasted_iota(jnp.int32, (RB, G), 1)
    onehot = (bvals[:, None] == gids).astype(jnp.float32)

    @pl.when(pl.program_id(0) == 0)
    def _():
        s_ref[...] = jnp.zeros_like(s_ref)
        c_ref[...] = jnp.zeros_like(c_ref)
        st_ref[...] = jnp.zeros_like(st_ref)

    dn = (((0,), (0,)), ((), ()))
    s_ref[...] += lax.dot_general(onehot, a, dn,
                                  preferred_element_type=jnp.float32)
    c_ref[...] += lax.dot_general(onehot, jnp.ones_like(a), dn,
                                  preferred_element_type=jnp.float32)
    st_ref[...] += jnp.concatenate(
        [jnp.sum(a, axis=0, keepdims=True),
         jnp.sum(a * a, axis=0, keepdims=True)], axis=0)


def _a2(Q, u2, degP, b2r, batch_r):
    return pl.pallas_call(
        _a2_kern,
        grid=(NB,),
        in_specs=[pl.BlockSpec((RB, H), lambda i: (i, 0)),
                  pl.BlockSpec((RB, H), lambda i: (i, 0)),
                  pl.BlockSpec((RB, 1), lambda i: (i, 0)),
                  pl.BlockSpec((1, H), lambda i: (0, 0)),
                  pl.BlockSpec((1, 1, RB), lambda i: (i, 0, 0))],
        out_specs=[pl.BlockSpec((G, H), lambda i: (0, 0)),
                   pl.BlockSpec((G, H), lambda i: (0, 0)),
                   pl.BlockSpec((2, H), lambda i: (0, 0))],
        out_shape=[jax.ShapeDtypeStruct((G, H), jnp.float32),
                   jax.ShapeDtypeStruct((G, H), jnp.float32),
                   jax.ShapeDtypeStruct((2, H), jnp.float32)],
    )(Q, u2, degP, b2r, batch_r)


def _final_kern(s_ref, c_ref, st_ref, g_ref, be_ref, wfc_ref, bfc_ref,
                o_ref):
    scale, shift = _bn_coeffs(st_ref, g_ref, be_ref)
    cnt = c_ref[...]
    pooled_raw = s_ref[...] / jnp.maximum(cnt, 1.0)
    pooled = jnp.where(cnt > 0.0, pooled_raw * scale + shift, 0.0)
    z = jnp.sum(pooled * wfc_ref[...], axis=1, keepdims=True) \
        + bfc_ref[0, 0]
    o_ref[...] = 1.0 / (1.0 + jnp.exp(-z))


def _final(S, C, stats2, g2r, be2r, wfcr, bfcr):
    return pl.pallas_call(
        _final_kern,
        in_specs=[pl.BlockSpec((G, H), lambda: (0, 0)),
                  pl.BlockSpec((G, H), lambda: (0, 0)),
                  pl.BlockSpec((2, H), lambda: (0, 0)),
                  pl.BlockSpec((1, H), lambda: (0, 0)),
                  pl.BlockSpec((1, H), lambda: (0, 0)),
                  pl.BlockSpec((1, H), lambda: (0, 0)),
                  pl.BlockSpec((1, 1), lambda: (0, 0))],
        out_specs=pl.BlockSpec((G, 1), lambda: (0, 0)),
        out_shape=jax.ShapeDtypeStruct((G, 1), jnp.float32),
    )(S, C, stats2, g2r, be2r, wfcr, bfcr)


def kernel(x, edge_index, batch, W1, b1, g1, be1, W2, b2, g2, be2, Wfc, bfc):
    src32 = edge_index[0].astype(jnp.int32)
    dst32 = edge_index[1].astype(jnp.int32)
    src_p = src32.reshape(NS, NCHP, CH)
    dst_p = dst32.reshape(NS, NCHP, CH)
    batch_r = batch.astype(jnp.int32).reshape(NB, 1, RB)
    b1r = b1.reshape(1, H)
    g1r = g1.reshape(1, H)
    be1r = be1.reshape(1, H)
    b2r = b2.reshape(1, H)
    g2r = g2.reshape(1, H)
    be2r = be2.reshape(1, H)
    wfcr = Wfc.reshape(1, H)
    bfcr = bfc.reshape(1, 1)

    zerosH = jnp.zeros((ZR, H), jnp.float32)
    onesD = jnp.ones((CH, DW), jnp.float32)
    zerosD = jnp.zeros((ZR, DW), jnp.float32)

    degc = _deg_sc(dst_p, onesD, zerosD).reshape(NC * NPAD2, DW)[:, :1]
    xW = _mm1(x, W1)
    u1 = _scaleu(degc, xW)
    P = _prop_sc(u1, src_p, dst_p, zerosH).reshape(NC * NPAD2, H)
    a1, stats1 = _a1(P, u1, degc, b1r)
    u2 = _m2(a1, degc, stats1, g1r, be1r, W2)
    Q = _prop_sc(u2, src_p, dst_p, zerosH).reshape(NC * NPAD2, H)
    S, C, stats2 = _a2(Q, u2, degc, b2r, batch_r)
    return _final(S, C, stats2, g2r, be2r, wfcr, bfcr)


# speedup vs baseline: 20.6773x; 1.1963x over previous
import functools

import jax
import jax.numpy as jnp
from jax import lax
from jax.experimental import pallas as pl
from jax.experimental.pallas import tpu as pltpu
from jax.experimental.pallas import tpu_sc as plsc

N = 10000
E = 320000
D = 128
H = 128
G = 64
EPS = 1e-5

NC = 2
NS = 16
NW = NC * NS
CH = 80
NCHP = E // NS // CH
ZR = 128

RB = 400
NB = N // RB

_sc_mesh = plsc.VectorSubcoreMesh(core_axis_name="c", subcore_axis_name="s",
                                  num_cores=NC, num_subcores=NS)


NPAD2 = 5120
RP2 = NPAD2 // NS
ACC_R = NPAD2 + NS
DW = 8
CHD = 80
NCHD = E // NS // CHD
SCH = 25


def _localize(dstv, nchunks, width, lo, garb):
    def body(j, _):
        for k in range(width // 16):
            v = dstv[j, pl.ds(k * 16, 16)]
            l = v - lo
            ok = (l >= 0) & (l < NPAD2)
            dstv[j, pl.ds(k * 16, 16)] = jnp.where(ok, l, garb)
        return 0

    lax.fori_loop(0, nchunks, body, 0)


def _deg_body(dst_hbm, ones_hbm, zeros_hbm, deg_hbm, dstv, onesv,
              acc_sh):
    c = lax.axis_index("c")
    s = lax.axis_index("s")

    pltpu.sync_copy(ones_hbm, onesv)
    base = s * RP2
    pltpu.sync_copy(zeros_hbm, acc_sh.at[pl.ds(base, ZR)])
    pltpu.sync_copy(zeros_hbm, acc_sh.at[pl.ds(base + ZR, ZR)])
    pltpu.sync_copy(zeros_hbm.at[pl.ds(0, RP2 - 2 * ZR)],
                    acc_sh.at[pl.ds(base + 2 * ZR, RP2 - 2 * ZR)])
    plsc.subcore_barrier()

    def stage(t, _):
        pltpu.sync_copy(dst_hbm.at[s, t], dstv)
        _localize(dstv, SCH, CHD, c * NPAD2, NPAD2 + s)

        def step(j, _):
            pltpu.sync_copy(onesv, acc_sh.at[dstv.at[j]], add=True)
            return 0

        lax.fori_loop(0, SCH, step, 0)
        return 0

    lax.fori_loop(0, NCHD // SCH, stage, 0)
    plsc.subcore_barrier()
    pltpu.sync_copy(acc_sh.at[pl.ds(base, RP2)],
                    deg_hbm.at[c, pl.ds(base, RP2)])


@functools.partial(
    pl.kernel,
    out_type=jax.ShapeDtypeStruct((NC, NPAD2, DW), jnp.float32),
    mesh=_sc_mesh,
    scratch_types=[
        pltpu.VMEM((SCH, CHD), jnp.int32),
        pltpu.VMEM((CHD, DW), jnp.float32),
        pltpu.VMEM_SHARED((ACC_R, DW), jnp.float32),
    ],
)
def _deg_sc(dst_hbm, ones_hbm, zeros_hbm, deg_hbm, dstv, onesv,
            acc_sh):
    _deg_body(dst_hbm, ones_hbm, zeros_hbm, deg_hbm, dstv, onesv,
              acc_sh)


NBUF = 5
NSP = NCHP // 2


def _prop_body(u_hbm, src_hbm, dst_hbm, zeros_hbm, out_hbm, srcv, dstv,
               rows, acc_sh, gsem):
    c = lax.axis_index("c")
    s = lax.axis_index("s")

    base = s * RP2
    pltpu.sync_copy(zeros_hbm, acc_sh.at[pl.ds(base, ZR)])
    pltpu.sync_copy(zeros_hbm, acc_sh.at[pl.ds(base + ZR, ZR)])
    pltpu.sync_copy(zeros_hbm.at[pl.ds(0, RP2 - 2 * ZR)],
                    acc_sh.at[pl.ds(base + 2 * ZR, RP2 - 2 * ZR)])

    for t in range(2):
        pltpu.sync_copy(src_hbm.at[s, t], srcv)
        pltpu.sync_copy(dst_hbm.at[s, t], dstv)
        _localize(dstv, NSP, CH, c * NPAD2, NPAD2 + s)
        if t == 0:
            plsc.subcore_barrier()

        for b in range(NBUF):
            pltpu.async_copy(u_hbm.at[srcv.at[b]], rows.at[b], gsem.at[b])

        def outer(g, _):
            for b in range(NBUF):
                j = g * NBUF + b
                pltpu.make_async_copy(u_hbm.at[srcv.at[j]], rows.at[b],
                                      gsem.at[b]).wait()
                pltpu.sync_copy(rows.at[b], acc_sh.at[dstv.at[j]],
                                add=True)
                nxt = j + NBUF

                @pl.when(nxt < NSP)
                def _():
                    pltpu.async_copy(u_hbm.at[srcv.at[nxt]], rows.at[b],
                                     gsem.at[b])
            return 0

        lax.fori_loop(0, NSP // NBUF, outer, 0)

    plsc.subcore_barrier()
    pltpu.sync_copy(acc_sh.at[pl.ds(s * RP2, RP2)],
                    out_hbm.at[c, pl.ds(s * RP2, RP2)])


@functools.partial(
    pl.kernel,
    out_type=jax.ShapeDtypeStruct((NC, NPAD2, H), jnp.float32),
    mesh=_sc_mesh,
    scratch_types=[
        pltpu.VMEM((NSP, CH), jnp.int32),
        pltpu.VMEM((NSP, CH), jnp.int32),
        pltpu.VMEM((NBUF, CH, H), jnp.float32),
        pltpu.VMEM_SHARED((ACC_R, H), jnp.float32),
        pltpu.SemaphoreType.DMA((NBUF,)),
    ],
)
def _prop_sc(u_hbm, src_hbm, dst_hbm, zeros_hbm, out_hbm, srcv, dstv,
             rows, acc_sh, gsem):
    _prop_body(u_hbm, src_hbm, dst_hbm, zeros_hbm, out_hbm, srcv, dstv,
               rows, acc_sh, gsem)


def _dinv_from_deg(degc_ref):
    deg = degc_ref[...] + 1.0
    return lax.rsqrt(jnp.maximum(deg, 1.0))


def _mm1_kern(x_ref, w_ref, o_ref):
    o_ref[...] = jnp.dot(x_ref[...], w_ref[...],
                         preferred_element_type=jnp.float32)


def _mm1(x, W1):
    return pl.pallas_call(
        _mm1_kern,
        grid=(NB,),
        in_specs=[pl.BlockSpec((RB, D), lambda i: (i, 0)),
                  pl.BlockSpec((D, H), lambda i: (0, 0))],
        out_specs=pl.BlockSpec((RB, H), lambda i: (i, 0)),
        out_shape=jax.ShapeDtypeStruct((N, H), jnp.float32),
    )(x, W1)


def _scaleu_kern(degp_ref, xw_ref, o_ref):
    o_ref[...] = xw_ref[...] * _dinv_from_deg(degp_ref)


def _scaleu(degP, xW):
    return pl.pallas_call(
        _scaleu_kern,
        grid=(NB,),
        in_specs=[pl.BlockSpec((RB, 1), lambda i: (i, 0)),
                  pl.BlockSpec((RB, H), lambda i: (i, 0))],
        out_specs=pl.BlockSpec((RB, H), lambda i: (i, 0)),
        out_shape=jax.ShapeDtypeStruct((N, H), jnp.float32),
    )(degP, xW)


def _a1_kern(p_ref, u_ref, degp_ref, b_ref, a_ref, st_ref):
    dinv = _dinv_from_deg(degp_ref)
    tot = p_ref[...] + u_ref[...]
    a = jnp.maximum(tot * dinv + b_ref[...], 0.0)
    a_ref[...] = a

    @pl.when(pl.program_id(0) == 0)
    def _():
        st_ref[...] = jnp.zeros_like(st_ref)

    st_ref[...] += jnp.concatenate(
        [jnp.sum(a, axis=0, keepdims=True),
         jnp.sum(a * a, axis=0, keepdims=True)], axis=0)


def _a1(P, u1, degP, b1r):
    return pl.pallas_call(
        _a1_kern,
        grid=(NB,),
        in_specs=[pl.BlockSpec((RB, H), lambda i: (i, 0)),
                  pl.BlockSpec((RB, H), lambda i: (i, 0)),
                  pl.BlockSpec((RB, 1), lambda i: (i, 0)),
                  pl.BlockSpec((1, H), lambda i: (0, 0))],
        out_specs=[pl.BlockSpec((RB, H), lambda i: (i, 0)),
                   pl.BlockSpec((2, H), lambda i: (0, 0))],
        out_shape=[jax.ShapeDtypeStruct((N, H), jnp.float32),
                   jax.ShapeDtypeStruct((2, H), jnp.float32)],
    )(P, u1, degP, b1r)


def _bn_coeffs(st_ref, g_ref, be_ref):
    mean = st_ref[0:1, :] * (1.0 / N)
    var = st_ref[1:2, :] * (1.0 / N) - mean * mean
    scale = g_ref[...] * lax.rsqrt(var + EPS)
    shift = be_ref[...] - mean * scale
    return scale, shift


def _m2_kern(a_ref, degp_ref, st_ref, g_ref, be_ref, w_ref, o_ref):
    scale, shift = _bn_coeffs(st_ref, g_ref, be_ref)
    t = a_ref[...] * scale + shift
    o_ref[...] = jnp.dot(t, w_ref[...],
                         preferred_element_type=jnp.float32) \
        * _dinv_from_deg(degp_ref)


def _m2(a1, degP, stats1, g1r, be1r, W2):
    return pl.pallas_call(
        _m2_kern,
        grid=(NB,),
        in_specs=[pl.BlockSpec((RB, H), lambda i: (i, 0)),
                  pl.BlockSpec((RB, 1), lambda i: (i, 0)),
                  pl.BlockSpec((2, H), lambda i: (0, 0)),
                  pl.BlockSpec((1, H), lambda i: (0, 0)),
                  pl.BlockSpec((1, H), lambda i: (0, 0)),
                  pl.BlockSpec((H, H), lambda i: (0, 0))],
        out_specs=pl.BlockSpec((RB, H), lambda i: (i, 0)),
        out_shape=jax.ShapeDtypeStruct((N, H), jnp.float32),
    )(a1, degP, stats1, g1r, be1r, W2)


def _a2_kern(q_ref, u_ref, degp_ref, b_ref, batch_ref,
             s_ref, c_ref, st_ref):
    dinv = _dinv_from_deg(degp_ref)
    tot = q_ref[...] + u_ref[...]
    a = jnp.maximum(tot * dinv + b_ref[...], 0.0)

    bvals = batch_ref[0, 0, :]
    gids = lax.broadcasted_iota(jnp.int32, (RB, G), 1)
    onehot = (bvals[:, None] == gids).astype(jnp.float32)

    @pl.when(pl.program_id(0) == 0)
    def _():
        s_ref[...] = jnp.zeros_like(s_ref)
        c_ref[...] = jnp.zeros_like(c_ref)
        st_ref[...] = jnp.zeros_like(st_ref)

    dn = (((0,), (0,)), ((), ()))
    s_ref[...] += lax.dot_general(onehot, a, dn,
                                  preferred_element_type=jnp.float32)
    c_ref[...] += lax.dot_general(onehot, jnp.ones_like(a), dn,
                                  preferred_element_type=jnp.float32)
    st_ref[...] += jnp.concatenate(
        [jnp.sum(a, axis=0, keepdims=True),
         jnp.sum(a * a, axis=0, keepdims=True)], axis=0)


def _a2(Q, u2, degP, b2r, batch_r):
    return pl.pallas_call(
        _a2_kern,
        grid=(NB,),
        in_specs=[pl.BlockSpec((RB, H), lambda i: (i, 0)),
                  pl.BlockSpec((RB, H), lambda i: (i, 0)),
                  pl.BlockSpec((RB, 1), lambda i: (i, 0)),
                  pl.BlockSpec((1, H), lambda i: (0, 0)),
                  pl.BlockSpec((1, 1, RB), lambda i: (i, 0, 0))],
        out_specs=[pl.BlockSpec((G, H), lambda i: (0, 0)),
                   pl.BlockSpec((G, H), lambda i: (0, 0)),
                   pl.BlockSpec((2, H), lambda i: (0, 0))],
        out_shape=[jax.ShapeDtypeStruct((G, H), jnp.float32),
                   jax.ShapeDtypeStruct((G, H), jnp.float32),
                   jax.ShapeDtypeStruct((2, H), jnp.float32)],
    )(Q, u2, degP, b2r, batch_r)


def _final_kern(s_ref, c_ref, st_ref, g_ref, be_ref, wfc_ref, bfc_ref,
                o_ref):
    scale, shift = _bn_coeffs(st_ref, g_ref, be_ref)
    cnt = c_ref[...]
    pooled_raw = s_ref[...] / jnp.maximum(cnt, 1.0)
    pooled = jnp.where(cnt > 0.0, pooled_raw * scale + shift, 0.0)
    z = jnp.sum(pooled * wfc_ref[...], axis=1, keepdims=True) \
        + bfc_ref[0, 0]
    o_ref[...] = 1.0 / (1.0 + jnp.exp(-z))


def _final(S, C, stats2, g2r, be2r, wfcr, bfcr):
    return pl.pallas_call(
        _final_kern,
        in_specs=[pl.BlockSpec((G, H), lambda: (0, 0)),
                  pl.BlockSpec((G, H), lambda: (0, 0)),
                  pl.BlockSpec((2, H), lambda: (0, 0)),
                  pl.BlockSpec((1, H), lambda: (0, 0)),
                  pl.BlockSpec((1, H), lambda: (0, 0)),
                  pl.BlockSpec((1, H), lambda: (0, 0)),
                  pl.BlockSpec((1, 1), lambda: (0, 0))],
        out_specs=pl.BlockSpec((G, 1), lambda: (0, 0)),
        out_shape=jax.ShapeDtypeStruct((G, 1), jnp.float32),
    )(S, C, stats2, g2r, be2r, wfcr, bfcr)


def kernel(x, edge_index, batch, W1, b1, g1, be1, W2, b2, g2, be2, Wfc, bfc):
    src32 = edge_index[0].astype(jnp.int32)
    dst32 = edge_index[1].astype(jnp.int32)
    src_p = src32.reshape(NS, 2, NSP, CH)
    dst_p = dst32.reshape(NS, 2, NSP, CH)
    dst_d = dst32.reshape(NS, NCHD // SCH, SCH, CHD)
    batch_r = batch.astype(jnp.int32).reshape(NB, 1, RB)
    b1r = b1.reshape(1, H)
    g1r = g1.reshape(1, H)
    be1r = be1.reshape(1, H)
    b2r = b2.reshape(1, H)
    g2r = g2.reshape(1, H)
    be2r = be2.reshape(1, H)
    wfcr = Wfc.reshape(1, H)
    bfcr = bfc.reshape(1, 1)

    zerosH = jnp.zeros((ZR, H), jnp.float32)
    onesD = jnp.ones((CHD, DW), jnp.float32)
    zerosD = jnp.zeros((ZR, DW), jnp.float32)

    degc = _deg_sc(dst_d, onesD, zerosD).reshape(NC * NPAD2, DW)[:, :1]
    xW = _mm1(x, W1)
    u1 = _scaleu(degc, xW)
    P = _prop_sc(u1, src_p, dst_p, zerosH).reshape(NC * NPAD2, H)
    a1, stats1 = _a1(P, u1, degc, b1r)
    u2 = _m2(a1, degc, stats1, g1r, be1r, W2)
    Q = _prop_sc(u2, src_p, dst_p, zerosH).reshape(NC * NPAD2, H)
    S, C, stats2 = _a2(Q, u2, degc, b2r, batch_r)
    return _final(S, C, stats2, g2r, be2r, wfcr, bfcr)
